# baseline (speedup 1.0000x reference)
import functools
import math

import jax
import jax.numpy as jnp
from jax import lax
from jax.experimental import pallas as pl
from jax.experimental.pallas import tpu as pltpu
from jax.experimental.pallas import tpu_sc as plsc

_B = 4
_C = 768
_T = 2048
_K = 384
_TAU = 1e-8
_L = 16
_NV = _C // _L


def _scores_body(x_ref, wq_ref, wk_ref, g_ref, n_ref):
    x = x_ref[0]
    s = jnp.mean(x, axis=1, keepdims=True)
    q = s * wq_ref[...]
    k = s * wk_ref[...]
    att = jax.lax.dot_general(
        q, k, (((1,), (1,)), ((), ())),
        preferred_element_type=jnp.float32) / math.sqrt(_C)
    att = jax.nn.softmax(att, axis=-1)
    scores = jnp.mean(att, axis=0, keepdims=True)
    noisy = scores + _TAU * g_ref[0]
    n_ref[0] = jax.lax.bitcast_convert_type(noisy, jnp.int32)


def _scores_tc(x, Wq, Wk, g):
    noisy = pl.pallas_call(
        _scores_body,
        grid=(_B,),
        in_specs=[
            pl.BlockSpec((1, _C, _T), lambda b: (b, 0, 0)),
            pl.BlockSpec((_C, _C), lambda b: (0, 0)),
            pl.BlockSpec((_C, _C), lambda b: (0, 0)),
            pl.BlockSpec((1, 1, _C), lambda b: (b, 0, 0)),
        ],
        out_specs=pl.BlockSpec((1, 1, _C), lambda b: (b, 0, 0)),
        out_shape=jax.ShapeDtypeStruct((_B, 1, _C), jnp.int32),
        compiler_params=pltpu.CompilerParams(
            dimension_semantics=("arbitrary",),
        ),
    )(x, Wq, Wk, g)
    return noisy.reshape(_B, _C)


def _select_body(noisy_hbm, mask_hbm, ibuf, mbuf, sbuf, sem):
    wid = lax.axis_index("s") * 2 + lax.axis_index("c")

    one_i = jnp.ones((_L,), jnp.int32)
    zero_i = jnp.zeros((_L,), jnp.int32)
    one_f = jnp.ones((_L,), jnp.float32)
    zero_f = jnp.zeros((_L,), jnp.float32)

    @pl.when(wid < _B)
    def _():
        pltpu.async_copy(noisy_hbm.at[wid], ibuf, sem).wait()
        sbuf[pl.ds(0, _L)] = zero_i
        sbuf[pl.ds(2 * _L, _L)] = zero_i

        def _total(x):
            for sh in (8, 4, 2, 1):
                sbuf[pl.ds(_L, _L)] = x
                x = x + sbuf[pl.ds(_L + sh, _L)]
            return x[0]

        def _count_ge(t):
            tv = jnp.broadcast_to(t, (_L,))
            acc = zero_i
            for v in range(_NV):
                acc = acc + jnp.where(
                    ibuf[pl.ds(v * _L, _L)] >= tv, one_i, zero_i)
            return _total(acc)

        def _bit_step(i, t):
            cand = t | (jnp.int32(1) << (jnp.int32(30) - i))
            return jnp.where(_count_ge(cand) >= _K, cand, t)

        thr = lax.fori_loop(0, 31, _bit_step, jnp.int32(0))
        thr_v = jnp.broadcast_to(thr, (_L,))
        acc = zero_i
        for v in range(_NV):
            acc = acc + jnp.where(
                ibuf[pl.ds(v * _L, _L)] > thr_v, one_i, zero_i)
        quota = _K - _total(acc)
        quota_v = jnp.broadcast_to(quota, (_L,))

        tie_seen = zero_i
        for v in range(_NV):
            iv = ibuf[pl.ds(v * _L, _L)]
            gt = iv > thr_v
            eq = iv == thr_v
            eq_i = jnp.where(eq, one_i, zero_i)
            ps = eq_i
            for sh in (1, 2, 4, 8):
                sbuf[pl.ds(_L, _L)] = ps
                ps = ps + sbuf[pl.ds(_L - sh, _L)]
            pref = ps - eq_i + tie_seen
            sel = gt | (eq & (pref < quota_v))
            mbuf[pl.ds(v * _L, _L)] = jnp.where(sel, one_f, zero_f)
            tie_seen = tie_seen + jnp.broadcast_to(ps[_L - 1], (_L,))
        pltpu.async_copy(mbuf, mask_hbm.at[wid], sem).wait()


def _select_sc(noisy):
    mesh = plsc.VectorSubcoreMesh(
        core_axis_name="c", subcore_axis_name="s", num_cores=2)
    kern = functools.partial(
        pl.kernel,
        mesh=mesh,
        out_type=jax.ShapeDtypeStruct((_B, _C), jnp.float32),
        scratch_types=[
            pltpu.VMEM((_C,), jnp.int32),
            pltpu.VMEM((_C,), jnp.float32),
            pltpu.VMEM((3 * _L,), jnp.int32),
            pltpu.SemaphoreType.DMA,
        ],
    )(_select_body)
    return kern(noisy)


def _apply_body(x_ref, m_ref, y_ref):
    y_ref[0] = x_ref[0] * m_ref[0]


def _apply_tc(x, mask):
    return pl.pallas_call(
        _apply_body,
        grid=(_B,),
        in_specs=[
            pl.BlockSpec((1, _C, _T), lambda b: (b, 0, 0)),
            pl.BlockSpec((1, _C, 1), lambda b: (b, 0, 0)),
        ],
        out_specs=pl.BlockSpec((1, _C, _T), lambda b: (b, 0, 0)),
        out_shape=jax.ShapeDtypeStruct((_B, _C, _T), jnp.float32),
        compiler_params=pltpu.CompilerParams(
            dimension_semantics=("arbitrary",),
        ),
    )(x, mask.reshape(_B, _C, 1))


def kernel(x, Wq, Wk):
    B, C, T = x.shape
    u = jax.random.uniform(jax.random.key(42), (B, C), minval=1e-20, maxval=1.0)
    g = (-jnp.log(-jnp.log(u))).reshape(B, 1, C)
    noisy = _scores_tc(x, Wq, Wk, g)
    mask = _select_sc(noisy)
    return _apply_tc(x, mask)


# speedup vs baseline: 2.5036x; 2.5036x over previous
import math

import jax
import jax.numpy as jnp
from jax.experimental import pallas as pl
from jax.experimental.pallas import tpu as pltpu

_C = 768
_T = 2048
_K = 384
_TAU = 1e-8


def _fused(x_ref, wq_ref, wk_ref, g_ref, y_ref):
    x = x_ref[0]
    s = jnp.mean(x, axis=1, keepdims=True)
    q = s * wq_ref[...]
    k = s * wk_ref[...]
    att = jax.lax.dot_general(
        q, k, (((1,), (1,)), ((), ())),
        preferred_element_type=jnp.float32) / math.sqrt(_C)
    att = jax.nn.softmax(att, axis=-1)
    scores = jnp.mean(att, axis=0, keepdims=True)
    noisy = scores + _TAU * g_ref[0]
    n_row = jnp.broadcast_to(noisy, (_C, _C))
    n_col = jnp.broadcast_to(noisy.reshape(_C, 1), (_C, _C))
    ii = jax.lax.broadcasted_iota(jnp.int32, (_C, _C), 0)
    jj = jax.lax.broadcasted_iota(jnp.int32, (_C, _C), 1)
    beats = (n_row > n_col) | ((n_row == n_col) & (jj < ii))
    rank = jnp.sum(beats.astype(jnp.int32), axis=1, keepdims=True)
    mask = (rank < _K).astype(jnp.float32)
    y_ref[0] = x * mask


def kernel(x, Wq, Wk):
    B, C, T = x.shape
    u = jax.random.uniform(jax.random.key(42), (B, C), minval=1e-20, maxval=1.0)
    g = (-jnp.log(-jnp.log(u))).reshape(B, 1, C)
    return pl.pallas_call(
        _fused,
        grid=(B,),
        in_specs=[
            pl.BlockSpec((1, _C, _T), lambda b: (b, 0, 0)),
            pl.BlockSpec((_C, _C), lambda b: (0, 0)),
            pl.BlockSpec((_C, _C), lambda b: (0, 0)),
            pl.BlockSpec((1, 1, _C), lambda b: (b, 0, 0)),
        ],
        out_specs=pl.BlockSpec((1, _C, _T), lambda b: (b, 0, 0)),
        out_shape=jax.ShapeDtypeStruct((B, C, T), jnp.float32),
        compiler_params=pltpu.CompilerParams(
            dimension_semantics=("parallel",),
        ),
    )(x, Wq, Wk, g)
